# Initial kernel scaffold; baseline (speedup 1.0000x reference)
import jax
import jax.numpy as jnp
from jax.experimental import pallas as pl


def kernel(x, edge_index, params):
    raise NotImplementedError("write your pallas kernel here")


# speedup vs baseline: 30.4376x; 30.4376x over previous
import functools

import jax
import jax.numpy as jnp
from jax import lax
from jax.experimental import pallas as pl
from jax.experimental.pallas import tpu as pltpu
from jax.experimental.pallas import tpu_sc as plsc

N_NODES = 10000
N_PAD = 10240
E_EDGES = 320000
K_CHUNK = 128
NWORK = 32
E_PAD = 323584
N_STEPS = E_PAD // (NWORK * K_CHUNK)
ROW_BLK = 512


def _dense_body(x, w_ref, as_ref, ad_ref, hs_ref, ad_out_ref, F):
    h = jnp.dot(x, w_ref[...], preferred_element_type=jnp.float32)
    a_s = jnp.dot(h, as_ref[...], preferred_element_type=jnp.float32)
    a_d = jnp.dot(h, ad_ref[...], preferred_element_type=jnp.float32)
    hs_ref[:, :F] = h
    hs_ref[:, F:] = a_s
    ad_out_ref[...] = a_d


def _first_dense_kernel(x_ref, w_ref, as_ref, ad_ref, hs_ref, ad_out_ref, *, F):
    _dense_body(x_ref[...], w_ref, as_ref, ad_ref, hs_ref, ad_out_ref, F)


def _norm_dense_kernel(acc_ref, b_ref, p_ref, w_ref, as_ref, ad_ref,
                       hs_ref, ad_out_ref, *, Fp, F):
    acc = acc_ref[0] + acc_ref[1]
    h_agg = acc[:, :Fp]
    den = acc[:, Fp:]
    denw = jnp.dot(den, p_ref[...], preferred_element_type=jnp.float32) + 1e-16
    x = jnp.maximum(h_agg / denw + b_ref[...], 0.0)
    _dense_body(x, w_ref, as_ref, ad_ref, hs_ref, ad_out_ref, F)


def _final_kernel(acc_ref, b_ref, out_ref):
    acc = acc_ref[0] + acc_ref[1]
    h_agg = acc[:, :128]
    sel = (lax.broadcasted_iota(jnp.int32, (16, 1), 0) == 0).astype(jnp.float32)
    den = jnp.dot(acc[:, 128:], sel, preferred_element_type=jnp.float32) + 1e-16
    out_ref[...] = jax.nn.sigmoid(h_agg / den + b_ref[...])


def _dense_call(x, w, as16, ad16, F):
    Fin = x.shape[1]
    G = F + 16
    grid = (N_PAD // ROW_BLK,)
    return pl.pallas_call(
        functools.partial(_first_dense_kernel, F=F),
        grid=grid,
        in_specs=[
            pl.BlockSpec((ROW_BLK, Fin), lambda i: (i, 0)),
            pl.BlockSpec((Fin, F), lambda i: (0, 0)),
            pl.BlockSpec((F, 16), lambda i: (0, 0)),
            pl.BlockSpec((F, 16), lambda i: (0, 0)),
        ],
        out_specs=[
            pl.BlockSpec((ROW_BLK, G), lambda i: (i, 0)),
            pl.BlockSpec((ROW_BLK, 16), lambda i: (i, 0)),
        ],
        out_shape=[
            jax.ShapeDtypeStruct((N_PAD, G), jnp.float32),
            jax.ShapeDtypeStruct((N_PAD, 16), jnp.float32),
        ],
    )(x, w, as16, ad16)


def _norm_dense_call(acc, b, p16, w, as16, ad16, Fp, F):
    Gp = Fp + 16
    G = F + 16
    grid = (N_PAD // ROW_BLK,)
    return pl.pallas_call(
        functools.partial(_norm_dense_kernel, Fp=Fp, F=F),
        grid=grid,
        in_specs=[
            pl.BlockSpec((2, ROW_BLK, Gp), lambda i: (0, i, 0)),
            pl.BlockSpec((1, Fp), lambda i: (0, 0)),
            pl.BlockSpec((16, Fp), lambda i: (0, 0)),
            pl.BlockSpec((Fp, F), lambda i: (0, 0)),
            pl.BlockSpec((F, 16), lambda i: (0, 0)),
            pl.BlockSpec((F, 16), lambda i: (0, 0)),
        ],
        out_specs=[
            pl.BlockSpec((ROW_BLK, G), lambda i: (i, 0)),
            pl.BlockSpec((ROW_BLK, 16), lambda i: (i, 0)),
        ],
        out_shape=[
            jax.ShapeDtypeStruct((N_PAD, G), jnp.float32),
            jax.ShapeDtypeStruct((N_PAD, 16), jnp.float32),
        ],
    )(acc, b, p16, w, as16, ad16)


def _final_call(acc, b):
    grid = (N_PAD // ROW_BLK,)
    return pl.pallas_call(
        _final_kernel,
        grid=grid,
        in_specs=[
            pl.BlockSpec((2, ROW_BLK, 144), lambda i: (0, i, 0)),
            pl.BlockSpec((1, 128), lambda i: (0, 0)),
        ],
        out_specs=pl.BlockSpec((ROW_BLK, 128), lambda i: (i, 0)),
        out_shape=jax.ShapeDtypeStruct((N_PAD, 128), jnp.float32),
    )(acc, b)


def _make_edge_kernel(F, H):
    G = F + 16
    C = F // H
    mesh = plsc.VectorSubcoreMesh(core_axis_name="c", subcore_axis_name="s")
    rows_per_sub = N_PAD // 16

    @functools.partial(
        pl.kernel,
        mesh=mesh,
        compiler_params=pltpu.CompilerParams(use_tc_tiling_on_sc=False),
        out_type=jax.ShapeDtypeStruct((2, N_PAD, G), jnp.float32),
        scratch_types=[
            pltpu.VMEM((K_CHUNK,), jnp.int32),
            pltpu.VMEM((K_CHUNK,), jnp.int32),
            pltpu.VMEM((K_CHUNK, G), jnp.float32),
            pltpu.VMEM((K_CHUNK, 16), jnp.float32),
            pltpu.VMEM_SHARED((N_PAD, G), jnp.float32),
            pltpu.SemaphoreType.DMA,
            pltpu.SemaphoreType.DMA,
        ],
    )
    def edge_kernel(src_hbm, dst_hbm, hs_hbm, ad_hbm, acc_hbm,
                    src_v, dst_v, rowsrc, adrows, acc_sh, sem1, sem2):
        cid = lax.axis_index("c")
        sid = lax.axis_index("s")
        wid = cid * 16 + sid

        def zrow(i, _):
            for j in range(G // 16):
                rowsrc[i, pl.ds(j * 16, 16)] = jnp.zeros((16,), jnp.float32)
            return 0
        lax.fori_loop(0, K_CHUNK, zrow, 0)
        for t in range(rows_per_sub // K_CHUNK):
            pltpu.sync_copy(
                rowsrc, acc_sh.at[pl.ds(sid * rows_per_sub + t * K_CHUNK, K_CHUNK)])
        plsc.subcore_barrier()

        def step(s, _):
            base = (s * NWORK + wid) * K_CHUNK
            pltpu.sync_copy(src_hbm.at[pl.ds(base, K_CHUNK)], src_v)
            pltpu.sync_copy(dst_hbm.at[pl.ds(base, K_CHUNK)], dst_v)
            cp1 = pltpu.async_copy(hs_hbm.at[src_v], rowsrc, sem1)
            cp2 = pltpu.async_copy(ad_hbm.at[dst_v], adrows, sem2)
            cp1.wait()
            cp2.wait()
            def mul_row(k, _):
                arow = rowsrc[k, pl.ds(F, 16)]
                drow = adrows[k, pl.ds(0, 16)]
                e = arow + drow
                e = jnp.maximum(e, 0.2 * e)
                w_row = jnp.exp(e)
                rowsrc[k, pl.ds(F, 16)] = w_row
                for hh in range(H):
                    s_w = w_row[hh]
                    for j in range(C // 16):
                        off = hh * C + j * 16
                        rowsrc[k, pl.ds(off, 16)] = rowsrc[k, pl.ds(off, 16)] * s_w
                return 0
            lax.fori_loop(0, K_CHUNK, mul_row, 0)
            pltpu.sync_copy(rowsrc, acc_sh.at[dst_v], add=True)
            return 0
        lax.fori_loop(0, N_STEPS, step, 0)

        plsc.subcore_barrier()
        pltpu.sync_copy(
            acc_sh.at[pl.ds(sid * rows_per_sub, rows_per_sub)],
            acc_hbm.at[cid, pl.ds(sid * rows_per_sub, rows_per_sub)])

    return edge_kernel


_EDGE_128_4 = _make_edge_kernel(128, 4)
_EDGE_32_1 = _make_edge_kernel(32, 1)
_EDGE_128_1 = _make_edge_kernel(128, 1)


def _attn_mats(p, heads, out_ch):
    F = heads * out_ch
    eye = jnp.eye(16, dtype=jnp.float32)[:heads]
    as16 = (p["a_src"][:, :, None] * eye[:, None, :]).reshape(F, 16)
    ad16 = (p["a_dst"][:, :, None] * eye[:, None, :]).reshape(F, 16)
    return as16, ad16


def _widen_mat(heads, out_ch):
    F = heads * out_ch
    eye = jnp.eye(16, dtype=jnp.float32)[:, :heads]
    return (eye[:, :, None] * jnp.ones((out_ch,), jnp.float32)).reshape(16, F)


def _edge_jnp(src, dst, hs, ad, F, H):
    G = F + 16
    C = F // H
    a_s = hs[:, F:]
    h = hs[:, :F]
    e = a_s[src] + ad[src * 0 + dst]
    e = jnp.maximum(e, 0.2 * e)
    w = jnp.exp(e)
    wh = jnp.repeat(w[:, :H], C, axis=1) * h[src]
    comb = jnp.concatenate([wh, w], axis=1)
    acc = jax.ops.segment_sum(comb, dst, num_segments=N_PAD)
    return jnp.stack([acc, jnp.zeros_like(acc)])


def kernel(x, edge_index, params):
    src = edge_index[0]
    dst = edge_index[1]
    x_pad = jnp.pad(x, ((0, N_PAD - N_NODES), (0, 0)))
    pad_e = E_PAD - E_EDGES
    src_p = jnp.concatenate([src, jnp.zeros((pad_e,), jnp.int32)])
    dst_p = jnp.concatenate([dst, jnp.full((pad_e,), N_NODES, jnp.int32)])

    p1, p2, p3, p4 = params["enc1"], params["enc2"], params["dec1"], params["dec2"]
    as1, ad1 = _attn_mats(p1, 4, 32)
    as2, ad2 = _attn_mats(p2, 1, 32)
    as3, ad3 = _attn_mats(p3, 4, 32)
    as4, ad4 = _attn_mats(p4, 1, 128)

    hs, ad = _dense_call(x_pad, p1["W"], as1, ad1, 128)
    acc = _EDGE_128_4(src_p, dst_p, hs, ad)

    hs, ad = _norm_dense_call(acc, p1["b"][None, :], _widen_mat(4, 32),
                              p2["W"], as2, ad2, 128, 32)
    acc = _EDGE_32_1(src_p, dst_p, hs, ad)

    hs, ad = _norm_dense_call(acc, p2["b"][None, :], _widen_mat(1, 32),
                              p3["W"], as3, ad3, 32, 128)
    acc = _EDGE_128_4(src_p, dst_p, hs, ad)

    hs, ad = _norm_dense_call(acc, p3["b"][None, :], _widen_mat(4, 32),
                              p4["W"], as4, ad4, 128, 128)
    acc = _EDGE_128_1(src_p, dst_p, hs, ad)

    out = _final_call(acc, p4["b"][None, :])
    return out[:N_NODES]
